# baseline (speedup 1.0000x reference)
import functools
import jax
import jax.numpy as jnp
from jax import lax
from jax.experimental import pallas as pl
from jax.experimental.pallas import tpu as pltpu
from jax.experimental.pallas import tpu_sc as plsc

BATCH = 64
EMBED_DIM = 768
NUM_PATCHES = 1024
NWORKERS = 32
E_PER_W = EMBED_DIM // NWORKERS
LANES = 16


def _tbody(pos_ref, out_ref):
    out_ref[...] = pos_ref[...].T


def _transpose_table(pos_table):
    return pl.pallas_call(
        _tbody,
        out_shape=jax.ShapeDtypeStruct((EMBED_DIM, NUM_PATCHES), jnp.float32),
    )(pos_table)


def _sc_body(patch_hbm, pos_t_hbm, out_hbm, pos_v, buf_v, sem):
    wid = lax.axis_index("s") * 2 + lax.axis_index("c")

    def e_loop(i, carry):
        e = wid * E_PER_W + i
        pltpu.sync_copy(pos_t_hbm.at[e], pos_v)
        pltpu.async_copy(patch_hbm.at[:, e, :], buf_v, sem).wait()

        def chunk_loop(j, c2):
            pv = pos_v[pl.ds(j * LANES, LANES)]

            def b_loop(b, c3):
                sl = pl.ds(j * LANES, LANES)
                buf_v[b, sl] = buf_v[b, sl] + pv
                return c3

            return lax.fori_loop(0, BATCH, b_loop, c2)

        lax.fori_loop(0, NUM_PATCHES // LANES, chunk_loop, carry)
        pltpu.sync_copy(buf_v, out_hbm.at[:, e, :])
        return carry

    lax.fori_loop(0, E_PER_W, e_loop, 0)


def kernel(patch, pos_table):
    pos_t = _transpose_table(pos_table)
    mesh = plsc.VectorSubcoreMesh(core_axis_name="c", subcore_axis_name="s")
    k = functools.partial(
        pl.kernel,
        mesh=mesh,
        out_type=jax.ShapeDtypeStruct((BATCH, EMBED_DIM, NUM_PATCHES), jnp.float32),
        scratch_types=[
            pltpu.VMEM((NUM_PATCHES,), jnp.float32),
            pltpu.VMEM((BATCH, NUM_PATCHES), jnp.float32),
            pltpu.SemaphoreType.DMA,
        ],
    )(_sc_body)
    return k(patch, pos_t)


# speedup vs baseline: 6.4441x; 6.4441x over previous
import jax
import jax.numpy as jnp
from jax.experimental import pallas as pl
from jax.experimental.pallas import tpu as pltpu

NUM_PATCHES = 1024
EMBED_DIM = 768
BATCH = 64


def _body(pos_ref, patch_ref, out_ref, tpos_ref):
    @pl.when(pl.program_id(0) == 0)
    def _init():
        tpos_ref[...] = pos_ref[...].T

    out_ref[...] = patch_ref[...] + tpos_ref[...][None, :, :]


BB = 4


def kernel(patch, pos_table):
    return pl.pallas_call(
        _body,
        grid=(BATCH // BB,),
        in_specs=[
            pl.BlockSpec((NUM_PATCHES, EMBED_DIM), lambda b: (0, 0)),
            pl.BlockSpec((BB, EMBED_DIM, NUM_PATCHES), lambda b: (b, 0, 0)),
        ],
        out_specs=pl.BlockSpec((BB, EMBED_DIM, NUM_PATCHES), lambda b: (b, 0, 0)),
        out_shape=jax.ShapeDtypeStruct((BATCH, EMBED_DIM, NUM_PATCHES), patch.dtype),
        scratch_shapes=[pltpu.VMEM((EMBED_DIM, NUM_PATCHES), jnp.float32)],
        compiler_params=pltpu.CompilerParams(
            dimension_semantics=("arbitrary",),
        ),
    )(pos_table, patch)
